# baseline (speedup 1.0000x reference)
import jax
import jax.numpy as jnp
from jax import lax
from jax.experimental import pallas as pl
from jax.experimental.pallas import tpu as pltpu
from jax.experimental.pallas import tpu_sc as plsc

BATCH = 16384
EMBED = 64
N_ROWS = 1000000
PAIR_ROWS = N_ROWS // 2
NC = 2
NS = 16
LANES = 16
NW = NC * NS
B_PER_W = BATCH // NW
CHUNK = 128
N_CHUNKS = B_PER_W // CHUNK
GROUPS = CHUNK // LANES

RELAYOUT_ROWS = 2000


def _relayout_body(lo_ref, hi_ref, out_ref):
    out_ref[:, 0:EMBED] = lo_ref[...]
    out_ref[:, EMBED:2 * EMBED] = hi_ref[...]


def _repack(table):
    nblk = PAIR_ROWS // RELAYOUT_ROWS
    return pl.pallas_call(
        _relayout_body,
        out_shape=jax.ShapeDtypeStruct((PAIR_ROWS, 2 * EMBED), jnp.float32),
        grid=(nblk,),
        in_specs=[
            pl.BlockSpec((RELAYOUT_ROWS, EMBED), lambda i: (i, 0)),
            pl.BlockSpec((RELAYOUT_ROWS, EMBED), lambda i, n=nblk: (i + n, 0)),
        ],
        out_specs=pl.BlockSpec((RELAYOUT_ROWS, 2 * EMBED), lambda i: (i, 0)),
    )(table, table)


def _sc_body(users_hbm, items_hbm, ut_hbm, it_hbm, out_hbm,
             uidx_v, iidx_v, upair_v, ipair_v, ubuf_v, ibuf_v, out_v,
             usem, isem):
    wid = lax.axis_index("s") * NC + lax.axis_index("c")
    base = wid * B_PER_W
    lane = lax.iota(jnp.int32, LANES)

    def chunk_body(ch, _):
        cbase = base + ch * CHUNK
        pltpu.sync_copy(users_hbm.at[pl.ds(cbase, CHUNK)], uidx_v)
        pltpu.sync_copy(items_hbm.at[pl.ds(cbase, CHUNK)], iidx_v)
        for t in range(CHUNK // LANES):
            sl = pl.ds(t * LANES, LANES)
            uv = uidx_v[sl]
            iv = iidx_v[sl]
            upair_v[sl] = uv - jnp.where(uv >= PAIR_ROWS, PAIR_ROWS, 0)
            ipair_v[sl] = iv - jnp.where(iv >= PAIR_ROWS, PAIR_ROWS, 0)
        cu = pltpu.async_copy(ut_hbm.at[upair_v], ubuf_v, usem)
        ci = pltpu.async_copy(it_hbm.at[ipair_v], ibuf_v, isem)
        cu.wait()
        ci.wait()
        for g in range(GROUPS):
            sl = pl.ds(g * LANES, LANES)
            uoff = jnp.where(uidx_v[sl] >= PAIR_ROWS, EMBED, 0)
            ioff = jnp.where(iidx_v[sl] >= PAIR_ROWS, EMBED, 0)
            rows = lane + g * LANES
            acc = jnp.zeros((LANES,), jnp.float32)
            ucol = uoff
            icol = ioff
            for _c in range(EMBED):
                u = plsc.load_gather(ubuf_v, [rows, ucol])
                it = plsc.load_gather(ibuf_v, [rows, icol])
                acc = acc + u * it
                ucol = ucol + 1
                icol = icol + 1
            res = 1.0 / (1.0 + jnp.exp(-acc))
            out_v[pl.ds(ch * CHUNK + g * LANES, LANES)] = res
        return 0

    lax.fori_loop(0, N_CHUNKS, chunk_body, 0)
    pltpu.sync_copy(out_v, out_hbm.at[pl.ds(base, B_PER_W)])


def _score(users, items, ut2, it2):
    mesh = plsc.VectorSubcoreMesh(core_axis_name="c", subcore_axis_name="s")
    k = pl.kernel(
        _sc_body,
        out_type=jax.ShapeDtypeStruct((BATCH,), jnp.float32),
        mesh=mesh,
        compiler_params=pltpu.CompilerParams(needs_layout_passes=False),
        scratch_types=[
            pltpu.VMEM((CHUNK,), jnp.int32),
            pltpu.VMEM((CHUNK,), jnp.int32),
            pltpu.VMEM((CHUNK,), jnp.int32),
            pltpu.VMEM((CHUNK,), jnp.int32),
            pltpu.VMEM((CHUNK, 2 * EMBED), jnp.float32),
            pltpu.VMEM((CHUNK, 2 * EMBED), jnp.float32),
            pltpu.VMEM((B_PER_W,), jnp.float32),
            pltpu.SemaphoreType.DMA,
            pltpu.SemaphoreType.DMA,
        ],
    )
    return k(users, items, ut2, it2)


@jax.jit
def kernel(users, items, user_table, item_table):
    ut2 = _repack(user_table)
    it2 = _repack(item_table)
    return _score(users, items, ut2, it2)


# speedup vs baseline: 1.8808x; 1.8808x over previous
import jax
import jax.numpy as jnp
from jax import lax
from jax.experimental import pallas as pl
from jax.experimental.pallas import tpu as pltpu
from jax.experimental.pallas import tpu_sc as plsc

BATCH = 16384
EMBED = 64
NC = 2
NS = 16
LANES = 16
NW = NC * NS
B_PER_W = BATCH // NW
CHUNK = 64
N_CHUNKS = B_PER_W // CHUNK
GROUPS = CHUNK // LANES


def _body(users_hbm, items_hbm, ut_hbm, it_hbm, out_hbm,
          uidx_v, iidx_v, ubuf_v, ibuf_v, out_v, sem):
    wid = lax.axis_index("s") * NC + lax.axis_index("c")
    base = wid * B_PER_W
    lane = lax.iota(jnp.int32, LANES)

    def chunk_body(ch, _):
        cbase = base + ch * CHUNK
        pltpu.sync_copy(users_hbm.at[pl.ds(cbase, CHUNK)], uidx_v)
        pltpu.sync_copy(items_hbm.at[pl.ds(cbase, CHUNK)], iidx_v)
        copies = []
        for g in range(GROUPS):
            uvec = uidx_v[pl.ds(g * LANES, LANES)]
            ivec = iidx_v[pl.ds(g * LANES, LANES)]
            for j in range(LANES):
                ru = jnp.sum(jnp.where(lane == j, uvec, 0))
                ri = jnp.sum(jnp.where(lane == j, ivec, 0))
                r = g * LANES + j
                copies.append(
                    pltpu.async_copy(ut_hbm.at[ru], ubuf_v.at[r], sem))
                copies.append(
                    pltpu.async_copy(it_hbm.at[ri], ibuf_v.at[r], sem))
        for c in copies:
            c.wait()
        for g in range(GROUPS):
            acc = jnp.zeros((LANES,), jnp.float32)
            for j in range(LANES):
                r = g * LANES + j
                p = jnp.zeros((LANES,), jnp.float32)
                for c in range(EMBED // LANES):
                    u = ubuf_v[r, pl.ds(c * LANES, LANES)]
                    it = ibuf_v[r, pl.ds(c * LANES, LANES)]
                    p = p + u * it
                s = jnp.sum(p)
                acc = jnp.where(lane == j, s, acc)
            res = 1.0 / (1.0 + jnp.exp(-acc))
            out_v[pl.ds(ch * CHUNK + g * LANES, LANES)] = res
        return 0

    lax.fori_loop(0, N_CHUNKS, chunk_body, 0)
    pltpu.sync_copy(out_v, out_hbm.at[pl.ds(base, B_PER_W)])


@jax.jit
def kernel(users, items, user_table, item_table):
    mesh = plsc.VectorSubcoreMesh(core_axis_name="c", subcore_axis_name="s")
    k = pl.kernel(
        _body,
        out_type=jax.ShapeDtypeStruct((BATCH,), jnp.float32),
        mesh=mesh,
        compiler_params=pltpu.CompilerParams(needs_layout_passes=False),
        scratch_types=[
            pltpu.VMEM((CHUNK,), jnp.int32),
            pltpu.VMEM((CHUNK,), jnp.int32),
            pltpu.VMEM((CHUNK, EMBED), jnp.float32),
            pltpu.VMEM((CHUNK, EMBED), jnp.float32),
            pltpu.VMEM((B_PER_W,), jnp.float32),
            pltpu.SemaphoreType.DMA,
        ],
    )
    return k(users, items, user_table, item_table)
